# initial kernel scaffold (unmeasured)
import jax
import jax.numpy as jnp
from jax import lax
from jax.experimental import pallas as pl
from jax.experimental.pallas import tpu as pltpu

N_DEV = 8


def kernel(x, w_mat, scale_x, scale_w):
    m, k_per = x.shape
    _, n = w_mat.shape
    m_per = m // N_DEV

    def body(x_ref, w_ref, sx_ref, sw_ref, out_ref,
             sbuf, rbuf, send_sems, recv_sems, credit_sem):
        d = lax.axis_index("i")
        left = lax.rem(d + N_DEV - 1, N_DEV)
        right = lax.rem(d + 1, N_DEV)

        barrier_sem = pltpu.get_barrier_semaphore()
        for nbr in (left, right):
            pl.semaphore_signal(
                barrier_sem, inc=1,
                device_id=(nbr,), device_id_type=pl.DeviceIdType.MESH,
            )
        pl.semaphore_wait(barrier_sem, 2)

        send_rdmas = [None, None]
        for s in range(N_DEV):
            j = lax.rem(d + (N_DEV - 1 - s), N_DEV)
            out_ref[...] = jnp.dot(
                x_ref[pl.ds(j * m_per, m_per), :], w_ref[...],
                preferred_element_type=jnp.float32,
            )
            if s >= 1:
                slot = (s - 1) % 2
                recv = pltpu.make_async_remote_copy(
                    src_ref=sbuf.at[0],
                    dst_ref=rbuf.at[slot],
                    send_sem=send_sems.at[0],
                    recv_sem=recv_sems.at[slot],
                    device_id=(left,), device_id_type=pl.DeviceIdType.MESH,
                )
                recv.wait_recv()
                out_ref[...] += rbuf[slot].astype(jnp.float32)
                if s <= N_DEV - 3:
                    pl.semaphore_signal(
                        credit_sem, inc=1,
                        device_id=(left,), device_id_type=pl.DeviceIdType.MESH,
                    )
            if s <= N_DEV - 2:
                slot = s % 2
                if send_rdmas[slot] is not None:
                    send_rdmas[slot].wait_send()
                sbuf[slot, :, :] = out_ref[...].astype(jnp.bfloat16)
                if s >= 2:
                    pl.semaphore_wait(credit_sem, 1)
                snd = pltpu.make_async_remote_copy(
                    src_ref=sbuf.at[slot],
                    dst_ref=rbuf.at[slot],
                    send_sem=send_sems.at[slot],
                    recv_sem=recv_sems.at[slot],
                    device_id=(right,), device_id_type=pl.DeviceIdType.MESH,
                )
                snd.start()
                send_rdmas[slot] = snd

        out_ref[...] = out_ref[...] * (sx_ref[0] * sw_ref[0])

        for slot in (0, 1):
            if send_rdmas[slot] is not None:
                send_rdmas[slot].wait_send()

    return pl.pallas_call(
        body,
        out_shape=jax.ShapeDtypeStruct((m_per, n), jnp.float32),
        in_specs=[
            pl.BlockSpec(memory_space=pltpu.VMEM),
            pl.BlockSpec(memory_space=pltpu.VMEM),
            pl.BlockSpec(memory_space=pltpu.SMEM),
            pl.BlockSpec(memory_space=pltpu.SMEM),
        ],
        out_specs=pl.BlockSpec(memory_space=pltpu.VMEM),
        scratch_shapes=[
            pltpu.VMEM((2, m_per, n), jnp.bfloat16),
            pltpu.VMEM((2, m_per, n), jnp.bfloat16),
            pltpu.SemaphoreType.DMA((2,)),
            pltpu.SemaphoreType.DMA((2,)),
            pltpu.SemaphoreType.REGULAR,
        ],
        compiler_params=pltpu.CompilerParams(collective_id=0),
    )(x, w_mat, scale_x, scale_w)


# baseline (device time: 700755 ns/iter reference)
import jax
import jax.numpy as jnp
from jax import lax
from jax.experimental import pallas as pl
from jax.experimental.pallas import tpu as pltpu

N_DEV = 8


def kernel(x, w_mat, scale_x, scale_w):
    m, k_per = x.shape
    _, n = w_mat.shape
    m_per = m // N_DEV

    x = x.astype(jnp.float8_e4m3fn)
    w_mat = w_mat.astype(jnp.float8_e5m2)

    def body(x_ref, w_ref, sx_ref, sw_ref, out_ref,
             sbuf, rbuf, send_sems, recv_sems, credit_sem):
        d = lax.axis_index("i")
        left = lax.rem(d + N_DEV - 1, N_DEV)
        right = lax.rem(d + 1, N_DEV)

        barrier_sem = pltpu.get_barrier_semaphore()
        for nbr in (left, right):
            pl.semaphore_signal(
                barrier_sem, inc=1,
                device_id=(nbr,), device_id_type=pl.DeviceIdType.MESH,
            )
        pl.semaphore_wait(barrier_sem, 2)

        send_rdmas = [None, None]
        for s in range(N_DEV):
            j = lax.rem(d + (N_DEV - 1 - s), N_DEV)
            out_ref[...] = jnp.dot(
                x_ref[pl.ds(j * m_per, m_per), :], w_ref[...],
                preferred_element_type=jnp.float32,
            )
            if s >= 1:
                slot = (s - 1) % 2
                recv = pltpu.make_async_remote_copy(
                    src_ref=sbuf.at[0],
                    dst_ref=rbuf.at[slot],
                    send_sem=send_sems.at[0],
                    recv_sem=recv_sems.at[slot],
                    device_id=(left,), device_id_type=pl.DeviceIdType.MESH,
                )
                recv.wait_recv()
                out_ref[...] += rbuf[slot].astype(jnp.float32)
                if s <= N_DEV - 3:
                    pl.semaphore_signal(
                        credit_sem, inc=1,
                        device_id=(left,), device_id_type=pl.DeviceIdType.MESH,
                    )
            if s <= N_DEV - 2:
                slot = s % 2
                if send_rdmas[slot] is not None:
                    send_rdmas[slot].wait_send()
                sbuf[slot, :, :] = out_ref[...].astype(jnp.bfloat16)
                if s >= 2:
                    pl.semaphore_wait(credit_sem, 1)
                snd = pltpu.make_async_remote_copy(
                    src_ref=sbuf.at[slot],
                    dst_ref=rbuf.at[slot],
                    send_sem=send_sems.at[slot],
                    recv_sem=recv_sems.at[slot],
                    device_id=(right,), device_id_type=pl.DeviceIdType.MESH,
                )
                snd.start()
                send_rdmas[slot] = snd

        out_ref[...] = out_ref[...] * (sx_ref[0] * sw_ref[0])

        for slot in (0, 1):
            if send_rdmas[slot] is not None:
                send_rdmas[slot].wait_send()

    return pl.pallas_call(
        body,
        out_shape=jax.ShapeDtypeStruct((m_per, n), jnp.float32),
        in_specs=[
            pl.BlockSpec(memory_space=pltpu.VMEM),
            pl.BlockSpec(memory_space=pltpu.VMEM),
            pl.BlockSpec(memory_space=pltpu.SMEM),
            pl.BlockSpec(memory_space=pltpu.SMEM),
        ],
        out_specs=pl.BlockSpec(memory_space=pltpu.VMEM),
        scratch_shapes=[
            pltpu.VMEM((2, m_per, n), jnp.bfloat16),
            pltpu.VMEM((2, m_per, n), jnp.bfloat16),
            pltpu.SemaphoreType.DMA((2,)),
            pltpu.SemaphoreType.DMA((2,)),
            pltpu.SemaphoreType.REGULAR,
        ],
        compiler_params=pltpu.CompilerParams(
            collective_id=0,
            vmem_limit_bytes=56 * 1024 * 1024,
        ),
    )(x, w_mat, scale_x, scale_w)


# device time: 379682 ns/iter; 1.8456x vs baseline; 1.8456x over previous
import jax
import jax.numpy as jnp
from jax import lax
from jax.experimental import pallas as pl
from jax.experimental.pallas import tpu as pltpu

N_DEV = 8


def kernel(x, w_mat, scale_x, scale_w):
    m, k_per = x.shape
    _, n = w_mat.shape
    m_per = m // N_DEV
    half = n // 2

    x = x.astype(jnp.float8_e4m3fn)
    w_mat = w_mat.astype(jnp.float8_e5m2)

    def body(x_ref, w_ref, sx_ref, sw_ref, out_ref,
             sbuf_r, rbuf_r, sbuf_l, rbuf_l,
             send_sems_r, recv_sems_r, send_sems_l, recv_sems_l,
             credit_r, credit_l):
        d = lax.axis_index("i")
        left = lax.rem(d + N_DEV - 1, N_DEV)
        right = lax.rem(d + 1, N_DEV)

        barrier_sem = pltpu.get_barrier_semaphore()
        for nbr in (left, right):
            pl.semaphore_signal(
                barrier_sem, inc=1,
                device_id=(nbr,), device_id_type=pl.DeviceIdType.MESH,
            )
        pl.semaphore_wait(barrier_sem, 2)

        send_rdmas = {"r": [None, None], "l": [None, None]}

        def ring_step(s, tag, cols, sbuf, rbuf, send_sems, recv_sems,
                      credit_sem, recv_from, send_to):
            if s >= 1:
                slot = (s - 1) % 2
                recv = pltpu.make_async_remote_copy(
                    src_ref=sbuf.at[0],
                    dst_ref=rbuf.at[slot],
                    send_sem=send_sems.at[0],
                    recv_sem=recv_sems.at[slot],
                    device_id=(recv_from,), device_id_type=pl.DeviceIdType.MESH,
                )
                recv.wait_recv()
                out_ref[:, cols] += rbuf[slot].astype(jnp.float32)
                if s <= N_DEV - 3:
                    pl.semaphore_signal(
                        credit_sem, inc=1,
                        device_id=(recv_from,),
                        device_id_type=pl.DeviceIdType.MESH,
                    )
            if s <= N_DEV - 2:
                slot = s % 2
                if send_rdmas[tag][slot] is not None:
                    send_rdmas[tag][slot].wait_send()
                sbuf[slot, :, :] = out_ref[:, cols].astype(jnp.bfloat16)
                if s >= 2:
                    pl.semaphore_wait(credit_sem, 1)
                snd = pltpu.make_async_remote_copy(
                    src_ref=sbuf.at[slot],
                    dst_ref=rbuf.at[slot],
                    send_sem=send_sems.at[slot],
                    recv_sem=recv_sems.at[slot],
                    device_id=(send_to,), device_id_type=pl.DeviceIdType.MESH,
                )
                snd.start()
                send_rdmas[tag][slot] = snd

        cols_r = pl.ds(0, half)
        cols_l = pl.ds(half, half)
        for s in range(N_DEV):
            jr = lax.rem(d + (N_DEV - 1 - s), N_DEV)
            jl = lax.rem(d + 1 + s, N_DEV)
            out_ref[:, cols_r] = jnp.dot(
                x_ref[pl.ds(jr * m_per, m_per), :], w_ref[:, cols_r],
                preferred_element_type=jnp.float32,
            )
            out_ref[:, cols_l] = jnp.dot(
                x_ref[pl.ds(jl * m_per, m_per), :], w_ref[:, cols_l],
                preferred_element_type=jnp.float32,
            )
            ring_step(s, "r", cols_r, sbuf_r, rbuf_r,
                      send_sems_r, recv_sems_r, credit_r, left, right)
            ring_step(s, "l", cols_l, sbuf_l, rbuf_l,
                      send_sems_l, recv_sems_l, credit_l, right, left)

        out_ref[...] = out_ref[...] * (sx_ref[0] * sw_ref[0])

        for tag in ("r", "l"):
            for slot in (0, 1):
                if send_rdmas[tag][slot] is not None:
                    send_rdmas[tag][slot].wait_send()

    return pl.pallas_call(
        body,
        out_shape=jax.ShapeDtypeStruct((m_per, n), jnp.float32),
        in_specs=[
            pl.BlockSpec(memory_space=pltpu.VMEM),
            pl.BlockSpec(memory_space=pltpu.VMEM),
            pl.BlockSpec(memory_space=pltpu.SMEM),
            pl.BlockSpec(memory_space=pltpu.SMEM),
        ],
        out_specs=pl.BlockSpec(memory_space=pltpu.VMEM),
        scratch_shapes=[
            pltpu.VMEM((2, m_per, half), jnp.bfloat16),
            pltpu.VMEM((2, m_per, half), jnp.bfloat16),
            pltpu.VMEM((2, m_per, half), jnp.bfloat16),
            pltpu.VMEM((2, m_per, half), jnp.bfloat16),
            pltpu.SemaphoreType.DMA((2,)),
            pltpu.SemaphoreType.DMA((2,)),
            pltpu.SemaphoreType.DMA((2,)),
            pltpu.SemaphoreType.DMA((2,)),
            pltpu.SemaphoreType.REGULAR,
            pltpu.SemaphoreType.REGULAR,
        ],
        compiler_params=pltpu.CompilerParams(
            collective_id=0,
            vmem_limit_bytes=56 * 1024 * 1024,
        ),
    )(x, w_mat, scale_x, scale_w)


# device time: 375470 ns/iter; 1.8663x vs baseline; 1.0112x over previous
import jax
import jax.numpy as jnp
from jax import lax
from jax.experimental import pallas as pl
from jax.experimental.pallas import tpu as pltpu

N_DEV = 8


def kernel(x, w_mat, scale_x, scale_w):
    m, k_per = x.shape
    _, n = w_mat.shape
    m_per = m // N_DEV
    half = n // 2

    x = x.astype(jnp.float8_e4m3fn)
    w_mat = w_mat.astype(jnp.float8_e5m2)

    def body(x_ref, w_ref, sx_ref, sw_ref, out_ref,
             sbuf_r, rbuf_r, sbuf_l, rbuf_l,
             send_sems_r, recv_sems_r, send_sems_l, recv_sems_l,
             credit_r, credit_l):
        d = lax.axis_index("i")
        left = lax.rem(d + N_DEV - 1, N_DEV)
        right = lax.rem(d + 1, N_DEV)

        barrier_sem = pltpu.get_barrier_semaphore()
        for nbr in (left, right):
            pl.semaphore_signal(
                barrier_sem, inc=1,
                device_id=(nbr,), device_id_type=pl.DeviceIdType.MESH,
            )
        pl.semaphore_wait(barrier_sem, 2)

        send_rdmas = {"r": [None, None], "l": [None, None]}

        def ring_step(s, tag, cols, sbuf, rbuf, send_sems, recv_sems,
                      credit_sem, recv_from, send_to):
            if s >= 1:
                slot = (s - 1) % 2
                recv = pltpu.make_async_remote_copy(
                    src_ref=sbuf.at[0],
                    dst_ref=rbuf.at[slot],
                    send_sem=send_sems.at[0],
                    recv_sem=recv_sems.at[slot],
                    device_id=(recv_from,), device_id_type=pl.DeviceIdType.MESH,
                )
                recv.wait_recv()
            if s == N_DEV - 1:
                out_ref[:, cols] = (
                    out_ref[:, cols] + rbuf[(s - 1) % 2].astype(jnp.float32)
                ) * (sx_ref[0] * sw_ref[0])
                return
            slot = s % 2
            if send_rdmas[tag][slot] is not None:
                send_rdmas[tag][slot].wait_send()
            if s == 0:
                sbuf[slot, :, :] = out_ref[:, cols].astype(jnp.bfloat16)
            else:
                sbuf[slot, :, :] = (
                    out_ref[:, cols] + rbuf[(s - 1) % 2].astype(jnp.float32)
                ).astype(jnp.bfloat16)
                if s <= N_DEV - 3:
                    pl.semaphore_signal(
                        credit_sem, inc=1,
                        device_id=(recv_from,),
                        device_id_type=pl.DeviceIdType.MESH,
                    )
            if s >= 2:
                pl.semaphore_wait(credit_sem, 1)
            snd = pltpu.make_async_remote_copy(
                src_ref=sbuf.at[slot],
                dst_ref=rbuf.at[slot],
                send_sem=send_sems.at[slot],
                recv_sem=recv_sems.at[slot],
                device_id=(send_to,), device_id_type=pl.DeviceIdType.MESH,
            )
            snd.start()
            send_rdmas[tag][slot] = snd

        cols_r = pl.ds(0, half)
        cols_l = pl.ds(half, half)
        for s in range(N_DEV):
            jr = lax.rem(d + (N_DEV - 1 - s), N_DEV)
            jl = lax.rem(d + 1 + s, N_DEV)
            out_ref[:, cols_r] = jnp.dot(
                x_ref[pl.ds(jr * m_per, m_per), :], w_ref[:, cols_r],
                preferred_element_type=jnp.float32,
            )
            out_ref[:, cols_l] = jnp.dot(
                x_ref[pl.ds(jl * m_per, m_per), :], w_ref[:, cols_l],
                preferred_element_type=jnp.float32,
            )
            ring_step(s, "r", cols_r, sbuf_r, rbuf_r,
                      send_sems_r, recv_sems_r, credit_r, left, right)
            ring_step(s, "l", cols_l, sbuf_l, rbuf_l,
                      send_sems_l, recv_sems_l, credit_l, right, left)

        for tag in ("r", "l"):
            for slot in (0, 1):
                if send_rdmas[tag][slot] is not None:
                    send_rdmas[tag][slot].wait_send()

    return pl.pallas_call(
        body,
        out_shape=jax.ShapeDtypeStruct((m_per, n), jnp.float32),
        in_specs=[
            pl.BlockSpec(memory_space=pltpu.VMEM),
            pl.BlockSpec(memory_space=pltpu.VMEM),
            pl.BlockSpec(memory_space=pltpu.SMEM),
            pl.BlockSpec(memory_space=pltpu.SMEM),
        ],
        out_specs=pl.BlockSpec(memory_space=pltpu.VMEM),
        scratch_shapes=[
            pltpu.VMEM((2, m_per, half), jnp.bfloat16),
            pltpu.VMEM((2, m_per, half), jnp.bfloat16),
            pltpu.VMEM((2, m_per, half), jnp.bfloat16),
            pltpu.VMEM((2, m_per, half), jnp.bfloat16),
            pltpu.SemaphoreType.DMA((2,)),
            pltpu.SemaphoreType.DMA((2,)),
            pltpu.SemaphoreType.DMA((2,)),
            pltpu.SemaphoreType.DMA((2,)),
            pltpu.SemaphoreType.REGULAR,
            pltpu.SemaphoreType.REGULAR,
        ],
        compiler_params=pltpu.CompilerParams(
            collective_id=0,
            vmem_limit_bytes=56 * 1024 * 1024,
        ),
    )(x, w_mat, scale_x, scale_w)


# device time: 360411 ns/iter; 1.9443x vs baseline; 1.0418x over previous
import jax
import jax.numpy as jnp
from jax import lax
from jax.experimental import pallas as pl
from jax.experimental.pallas import tpu as pltpu

N_DEV = 8
N_Q = 2


def kernel(x, w_mat, scale_x, scale_w):
    m, k_per = x.shape
    _, n = w_mat.shape
    m_per = m // N_DEV
    half = n // 2
    qn = half // N_Q

    x = x.astype(jnp.float8_e4m3fn)
    w_mat = w_mat.astype(jnp.float8_e5m2)

    def body(x_ref, w_ref, sx_ref, sw_ref, out_ref,
             sbuf_r, rbuf_r, sbuf_l, rbuf_l,
             send_sems_r, recv_sems_r, send_sems_l, recv_sems_l,
             credit_r, credit_l):
        d = lax.axis_index("i")
        left = lax.rem(d + N_DEV - 1, N_DEV)
        right = lax.rem(d + 1, N_DEV)

        barrier_sem = pltpu.get_barrier_semaphore()
        for nbr in (left, right):
            pl.semaphore_signal(
                barrier_sem, inc=1,
                device_id=(nbr,), device_id_type=pl.DeviceIdType.MESH,
            )
        pl.semaphore_wait(barrier_sem, 2)

        send_rdmas = {}

        def quarter_step(s, tag, q, base, sbuf, rbuf, send_sems, recv_sems,
                         credit_sem, recv_from, send_to):
            cols = pl.ds(base + q * qn, qn)
            pslot = (s - 1) % 2
            if s >= 1:
                recv = pltpu.make_async_remote_copy(
                    src_ref=sbuf.at[0, 0],
                    dst_ref=rbuf.at[pslot, q],
                    send_sem=send_sems.at[0, 0],
                    recv_sem=recv_sems.at[pslot, q],
                    device_id=(recv_from,), device_id_type=pl.DeviceIdType.MESH,
                )
                recv.wait_recv()
            if s == N_DEV - 1:
                out_ref[:, cols] = (
                    out_ref[:, cols] + rbuf[pslot, q].astype(jnp.float32)
                ) * (sx_ref[0] * sw_ref[0])
                return
            slot = s % 2
            key = (tag, slot, q)
            if key in send_rdmas:
                send_rdmas[key].wait_send()
            if s == 0:
                sbuf[slot, q] = out_ref[:, cols].astype(jnp.bfloat16)
            else:
                sbuf[slot, q] = (
                    out_ref[:, cols] + rbuf[pslot, q].astype(jnp.float32)
                ).astype(jnp.bfloat16)
                if s <= N_DEV - 3:
                    pl.semaphore_signal(
                        credit_sem, inc=1,
                        device_id=(recv_from,),
                        device_id_type=pl.DeviceIdType.MESH,
                    )
            if s >= 2:
                pl.semaphore_wait(credit_sem, 1)
            snd = pltpu.make_async_remote_copy(
                src_ref=sbuf.at[slot, q],
                dst_ref=rbuf.at[slot, q],
                send_sem=send_sems.at[slot, q],
                recv_sem=recv_sems.at[slot, q],
                device_id=(send_to,), device_id_type=pl.DeviceIdType.MESH,
            )
            snd.start()
            send_rdmas[key] = snd

        for s in range(N_DEV):
            jr = lax.rem(d + (N_DEV - 1 - s), N_DEV)
            jl = lax.rem(d + 1 + s, N_DEV)
            out_ref[:, pl.ds(0, half)] = jnp.dot(
                x_ref[pl.ds(jr * m_per, m_per), :], w_ref[:, pl.ds(0, half)],
                preferred_element_type=jnp.float32,
            )
            out_ref[:, pl.ds(half, half)] = jnp.dot(
                x_ref[pl.ds(jl * m_per, m_per), :], w_ref[:, pl.ds(half, half)],
                preferred_element_type=jnp.float32,
            )
            for q in range(N_Q):
                quarter_step(s, "r", q, 0, sbuf_r, rbuf_r,
                             send_sems_r, recv_sems_r, credit_r, left, right)
                quarter_step(s, "l", q, half, sbuf_l, rbuf_l,
                             send_sems_l, recv_sems_l, credit_l, right, left)

        for snd in send_rdmas.values():
            snd.wait_send()

    return pl.pallas_call(
        body,
        out_shape=jax.ShapeDtypeStruct((m_per, n), jnp.float32),
        in_specs=[
            pl.BlockSpec(memory_space=pltpu.VMEM),
            pl.BlockSpec(memory_space=pltpu.VMEM),
            pl.BlockSpec(memory_space=pltpu.SMEM),
            pl.BlockSpec(memory_space=pltpu.SMEM),
        ],
        out_specs=pl.BlockSpec(memory_space=pltpu.VMEM),
        scratch_shapes=[
            pltpu.VMEM((2, N_Q, m_per, qn), jnp.bfloat16),
            pltpu.VMEM((2, N_Q, m_per, qn), jnp.bfloat16),
            pltpu.VMEM((2, N_Q, m_per, qn), jnp.bfloat16),
            pltpu.VMEM((2, N_Q, m_per, qn), jnp.bfloat16),
            pltpu.SemaphoreType.DMA((2, N_Q)),
            pltpu.SemaphoreType.DMA((2, N_Q)),
            pltpu.SemaphoreType.DMA((2, N_Q)),
            pltpu.SemaphoreType.DMA((2, N_Q)),
            pltpu.SemaphoreType.REGULAR,
            pltpu.SemaphoreType.REGULAR,
        ],
        compiler_params=pltpu.CompilerParams(
            collective_id=0,
            vmem_limit_bytes=56 * 1024 * 1024,
        ),
    )(x, w_mat, scale_x, scale_w)


# device time: 360246 ns/iter; 1.9452x vs baseline; 1.0005x over previous
import jax
import jax.numpy as jnp
from jax import lax
from jax.experimental import pallas as pl
from jax.experimental.pallas import tpu as pltpu

N_DEV = 8
N_Q = 4


def kernel(x, w_mat, scale_x, scale_w):
    m, k_per = x.shape
    _, n = w_mat.shape
    m_per = m // N_DEV
    half = n // 2
    qn = half // N_Q

    x = x.astype(jnp.float8_e4m3fn)
    w_mat = w_mat.astype(jnp.float8_e5m2)

    def body(x_ref, w_ref, sx_ref, sw_ref, out_ref,
             sbuf_r, rbuf_r, sbuf_l, rbuf_l,
             send_sems_r, recv_sems_r, send_sems_l, recv_sems_l,
             credit_r, credit_l):
        d = lax.axis_index("i")
        left = lax.rem(d + N_DEV - 1, N_DEV)
        right = lax.rem(d + 1, N_DEV)

        barrier_sem = pltpu.get_barrier_semaphore()
        for nbr in (left, right):
            pl.semaphore_signal(
                barrier_sem, inc=1,
                device_id=(nbr,), device_id_type=pl.DeviceIdType.MESH,
            )
        pl.semaphore_wait(barrier_sem, 2)

        send_rdmas = {}

        def quarter_step(s, tag, q, base, sbuf, rbuf, send_sems, recv_sems,
                         credit_sem, recv_from, send_to):
            cols = pl.ds(base + q * qn, qn)
            pslot = (s - 1) % 2
            if s >= 1:
                recv = pltpu.make_async_remote_copy(
                    src_ref=sbuf.at[0, 0],
                    dst_ref=rbuf.at[pslot, q],
                    send_sem=send_sems.at[0, 0],
                    recv_sem=recv_sems.at[pslot, q],
                    device_id=(recv_from,), device_id_type=pl.DeviceIdType.MESH,
                )
                recv.wait_recv()
            if s == N_DEV - 1:
                out_ref[:, cols] = (
                    out_ref[:, cols] + rbuf[pslot, q].astype(jnp.float32)
                ) * (sx_ref[0] * sw_ref[0])
                return
            slot = s % 2
            key = (tag, slot, q)
            if key in send_rdmas:
                send_rdmas[key].wait_send()
            if s == 0:
                sbuf[slot, q] = out_ref[:, cols].astype(jnp.bfloat16)
            else:
                sbuf[slot, q] = (
                    out_ref[:, cols] + rbuf[pslot, q].astype(jnp.float32)
                ).astype(jnp.bfloat16)
                if s <= N_DEV - 3:
                    pl.semaphore_signal(
                        credit_sem, inc=1,
                        device_id=(recv_from,),
                        device_id_type=pl.DeviceIdType.MESH,
                    )
            if s >= 2:
                pl.semaphore_wait(credit_sem, 1)
            snd = pltpu.make_async_remote_copy(
                src_ref=sbuf.at[slot, q],
                dst_ref=rbuf.at[slot, q],
                send_sem=send_sems.at[slot, q],
                recv_sem=recv_sems.at[slot, q],
                device_id=(send_to,), device_id_type=pl.DeviceIdType.MESH,
            )
            snd.start()
            send_rdmas[key] = snd

        for s in range(N_DEV):
            jr = lax.rem(d + (N_DEV - 1 - s), N_DEV)
            jl = lax.rem(d + 1 + s, N_DEV)
            out_ref[:, pl.ds(0, half)] = jnp.dot(
                x_ref[pl.ds(jr * m_per, m_per), :], w_ref[:, pl.ds(0, half)],
                preferred_element_type=jnp.float32,
            )
            out_ref[:, pl.ds(half, half)] = jnp.dot(
                x_ref[pl.ds(jl * m_per, m_per), :], w_ref[:, pl.ds(half, half)],
                preferred_element_type=jnp.float32,
            )
            for q in range(N_Q):
                quarter_step(s, "r", q, 0, sbuf_r, rbuf_r,
                             send_sems_r, recv_sems_r, credit_r, left, right)
                quarter_step(s, "l", q, half, sbuf_l, rbuf_l,
                             send_sems_l, recv_sems_l, credit_l, right, left)

        for snd in send_rdmas.values():
            snd.wait_send()

    return pl.pallas_call(
        body,
        out_shape=jax.ShapeDtypeStruct((m_per, n), jnp.float32),
        in_specs=[
            pl.BlockSpec(memory_space=pltpu.VMEM),
            pl.BlockSpec(memory_space=pltpu.VMEM),
            pl.BlockSpec(memory_space=pltpu.SMEM),
            pl.BlockSpec(memory_space=pltpu.SMEM),
        ],
        out_specs=pl.BlockSpec(memory_space=pltpu.VMEM),
        scratch_shapes=[
            pltpu.VMEM((2, N_Q, m_per, qn), jnp.bfloat16),
            pltpu.VMEM((2, N_Q, m_per, qn), jnp.bfloat16),
            pltpu.VMEM((2, N_Q, m_per, qn), jnp.bfloat16),
            pltpu.VMEM((2, N_Q, m_per, qn), jnp.bfloat16),
            pltpu.SemaphoreType.DMA((2, N_Q)),
            pltpu.SemaphoreType.DMA((2, N_Q)),
            pltpu.SemaphoreType.DMA((2, N_Q)),
            pltpu.SemaphoreType.DMA((2, N_Q)),
            pltpu.SemaphoreType.REGULAR,
            pltpu.SemaphoreType.REGULAR,
        ],
        compiler_params=pltpu.CompilerParams(
            collective_id=0,
            vmem_limit_bytes=56 * 1024 * 1024,
        ),
    )(x, w_mat, scale_x, scale_w)


# device time: 344303 ns/iter; 2.0353x vs baseline; 1.0463x over previous
import jax
import jax.numpy as jnp
from jax import lax
from jax.experimental import pallas as pl
from jax.experimental.pallas import tpu as pltpu

N_DEV = 8
N_Q = 4


def kernel(x, w_mat, scale_x, scale_w):
    m, k_per = x.shape
    _, n = w_mat.shape
    m_per = m // N_DEV
    half = n // 2
    qn = half // N_Q
    xc = 4
    xrows = m // xc

    def body(x_ref, w_ref, sx_ref, sw_ref, out_ref,
             x8, w8, xs, ws,
             sbuf_r, rbuf_r, sbuf_l, rbuf_l,
             xsems, wsems,
             send_sems_r, recv_sems_r, send_sems_l, recv_sems_l,
             credit_r, credit_l):
        d = lax.axis_index("i")
        left = lax.rem(d + N_DEV - 1, N_DEV)
        right = lax.rem(d + 1, N_DEV)

        worder = []
        for q in range(N_Q):
            worder += [q, N_Q + q]

        def xdma(k, slot):
            return pltpu.make_async_copy(
                x_ref.at[pl.ds(k * xrows, xrows), :], xs.at[slot],
                xsems.at[slot],
            )

        def wdma(c, slot):
            return pltpu.make_async_copy(
                w_ref.at[:, pl.ds(c * qn, qn)], ws.at[slot],
                wsems.at[slot],
            )

        xdma(0, 0).start()
        xdma(1, 1).start()
        wdma(worder[0], 0).start()
        wdma(worder[1], 1).start()

        barrier_sem = pltpu.get_barrier_semaphore()
        for nbr in (left, right):
            pl.semaphore_signal(
                barrier_sem, inc=1,
                device_id=(nbr,), device_id_type=pl.DeviceIdType.MESH,
            )
        pl.semaphore_wait(barrier_sem, 2)

        for k in range(xc):
            xdma(k, k % 2).wait()
            x8[pl.ds(k * xrows, xrows), :] = xs[k % 2].astype(
                jnp.float8_e4m3fn)
            if k + 2 < xc:
                xdma(k + 2, k % 2).start()

        def wconvert(idx):
            c = worder[idx]
            wdma(c, idx % 2).wait()
            w8[:, pl.ds(c * qn, qn)] = ws[idx % 2].astype(jnp.float8_e5m2)
            if idx + 2 < 2 * N_Q:
                wdma(worder[idx + 2], idx % 2).start()

        send_rdmas = {}

        def quarter_step(s, tag, q, base, sbuf, rbuf, send_sems, recv_sems,
                         credit_sem, recv_from, send_to):
            cols = pl.ds(base + q * qn, qn)
            pslot = (s - 1) % 2
            if s >= 1:
                recv = pltpu.make_async_remote_copy(
                    src_ref=sbuf.at[0],
                    dst_ref=rbuf.at[pslot, q],
                    send_sem=send_sems.at[0],
                    recv_sem=recv_sems.at[pslot, q],
                    device_id=(recv_from,), device_id_type=pl.DeviceIdType.MESH,
                )
                recv.wait_recv()
            if s == N_DEV - 1:
                out_ref[:, cols] = (
                    out_ref[:, cols] + rbuf[pslot, q].astype(jnp.float32)
                ) * (sx_ref[0] * sw_ref[0])
                return
            key = (tag, q)
            if key in send_rdmas:
                send_rdmas[key].wait_send()
            if s == 0:
                sbuf[q] = out_ref[:, cols].astype(jnp.bfloat16)
            else:
                sbuf[q] = (
                    out_ref[:, cols] + rbuf[pslot, q].astype(jnp.float32)
                ).astype(jnp.bfloat16)
                if s <= N_DEV - 3:
                    pl.semaphore_signal(
                        credit_sem, inc=1,
                        device_id=(recv_from,),
                        device_id_type=pl.DeviceIdType.MESH,
                    )
            if s >= 2:
                pl.semaphore_wait(credit_sem, 1)
            snd = pltpu.make_async_remote_copy(
                src_ref=sbuf.at[q],
                dst_ref=rbuf.at[s % 2, q],
                send_sem=send_sems.at[q],
                recv_sem=recv_sems.at[s % 2, q],
                device_id=(send_to,), device_id_type=pl.DeviceIdType.MESH,
            )
            snd.start()
            send_rdmas[key] = snd

        for s in range(N_DEV):
            jr = lax.rem(d + (N_DEV - 1 - s), N_DEV)
            jl = lax.rem(d + 1 + s, N_DEV)
            if s == 0:
                for q in range(N_Q):
                    cols_r = pl.ds(q * qn, qn)
                    cols_l = pl.ds(half + q * qn, qn)
                    wconvert(2 * q)
                    out_ref[:, cols_r] = jnp.dot(
                        x8[pl.ds(jr * m_per, m_per), :], w8[:, cols_r],
                        preferred_element_type=jnp.float32,
                    )
                    quarter_step(s, "r", q, 0, sbuf_r, rbuf_r,
                                 send_sems_r, recv_sems_r, credit_r,
                                 left, right)
                    wconvert(2 * q + 1)
                    out_ref[:, cols_l] = jnp.dot(
                        x8[pl.ds(jl * m_per, m_per), :], w8[:, cols_l],
                        preferred_element_type=jnp.float32,
                    )
                    quarter_step(s, "l", q, half, sbuf_l, rbuf_l,
                                 send_sems_l, recv_sems_l, credit_l,
                                 right, left)
                continue
            out_ref[:, pl.ds(0, half)] = jnp.dot(
                x8[pl.ds(jr * m_per, m_per), :], w8[:, pl.ds(0, half)],
                preferred_element_type=jnp.float32,
            )
            out_ref[:, pl.ds(half, half)] = jnp.dot(
                x8[pl.ds(jl * m_per, m_per), :], w8[:, pl.ds(half, half)],
                preferred_element_type=jnp.float32,
            )
            for q in range(N_Q):
                quarter_step(s, "r", q, 0, sbuf_r, rbuf_r,
                             send_sems_r, recv_sems_r, credit_r, left, right)
                quarter_step(s, "l", q, half, sbuf_l, rbuf_l,
                             send_sems_l, recv_sems_l, credit_l, right, left)

        for snd in send_rdmas.values():
            snd.wait_send()

    return pl.pallas_call(
        body,
        out_shape=jax.ShapeDtypeStruct((m_per, n), jnp.float32),
        in_specs=[
            pl.BlockSpec(memory_space=pl.ANY),
            pl.BlockSpec(memory_space=pl.ANY),
            pl.BlockSpec(memory_space=pltpu.SMEM),
            pl.BlockSpec(memory_space=pltpu.SMEM),
        ],
        out_specs=pl.BlockSpec(memory_space=pltpu.VMEM),
        scratch_shapes=[
            pltpu.VMEM((m, k_per), jnp.float8_e4m3fn),
            pltpu.VMEM((k_per, n), jnp.float8_e5m2),
            pltpu.VMEM((2, m // 4, k_per), jnp.float32),
            pltpu.VMEM((2, k_per, qn), jnp.float32),
            pltpu.VMEM((N_Q, m_per, qn), jnp.bfloat16),
            pltpu.VMEM((2, N_Q, m_per, qn), jnp.bfloat16),
            pltpu.VMEM((N_Q, m_per, qn), jnp.bfloat16),
            pltpu.VMEM((2, N_Q, m_per, qn), jnp.bfloat16),
            pltpu.SemaphoreType.DMA((2,)),
            pltpu.SemaphoreType.DMA((2,)),
            pltpu.SemaphoreType.DMA((N_Q,)),
            pltpu.SemaphoreType.DMA((2, N_Q)),
            pltpu.SemaphoreType.DMA((N_Q,)),
            pltpu.SemaphoreType.DMA((2, N_Q)),
            pltpu.SemaphoreType.REGULAR,
            pltpu.SemaphoreType.REGULAR,
        ],
        compiler_params=pltpu.CompilerParams(
            collective_id=0,
            vmem_limit_bytes=60 * 1024 * 1024,
        ),
    )(x, w_mat, scale_x, scale_w)
